# baseline (speedup 1.0000x reference)
import functools

import jax
import jax.numpy as jnp
from jax import lax
from jax.experimental import pallas as pl
from jax.experimental.pallas import tpu as pltpu
from jax.experimental.pallas import tpu_sc as plsc

VOCAB = 1000
DIM = 1000
HALF = 512
DIM_PK = 512
BATCH = 16384

_PARTS = (8192, 8192)

_TBL_ROWS = 200


def _pack_body(t_ref, o_ref):
    t = t_ref[...]
    m = jnp.max(t, axis=1, keepdims=True)
    e = jnp.exp(t - m)
    sm = e / jnp.sum(e, axis=1, keepdims=True)
    lo = sm[:, :HALF]
    hi = jnp.concatenate(
        [sm[:, HALF:], jnp.zeros((_TBL_ROWS, 2 * HALF - DIM), jnp.float32)], axis=1
    )
    lo_b = lax.bitcast_convert_type(lo, jnp.uint32)
    hi_b = lax.bitcast_convert_type(hi, jnp.uint32)
    half = jnp.uint32(0x8000)
    word = ((hi_b + half) & jnp.uint32(0xFFFF0000)) | ((lo_b + half) >> 16)
    o_ref[...] = lax.bitcast_convert_type(word, jnp.float32)


def _pack_table(table):
    return pl.pallas_call(
        _pack_body,
        grid=(VOCAB // _TBL_ROWS,),
        in_specs=[pl.BlockSpec((_TBL_ROWS, DIM), lambda i: (i, 0))],
        out_specs=pl.BlockSpec((_TBL_ROWS, DIM_PK), lambda i: (i, 0)),
        out_shape=jax.ShapeDtypeStruct((VOCAB, DIM_PK), jnp.float32),
    )(table)


_NC = 2
_NS = 16
_NW = _NC * _NS
_CHUNK = 64
_NBUF = 3

_sc_mesh = plsc.VectorSubcoreMesh(core_axis_name="c", subcore_axis_name="s")


@functools.cache
def _make_gather(bp):
    b_per_w = bp // _NW
    nchunk = b_per_w // _CHUNK

    @functools.partial(
        pl.kernel,
        out_type=jax.ShapeDtypeStruct((bp, DIM_PK), jnp.float32),
        mesh=_sc_mesh,
        scratch_types=[
            pltpu.VMEM((b_per_w,), jnp.int32),
            pltpu.VMEM((_NBUF, _CHUNK, DIM_PK), jnp.float32),
            pltpu.SemaphoreType.DMA,
            pltpu.SemaphoreType.DMA,
            pltpu.SemaphoreType.DMA,
        ],
    )
    def _gather_rows(table_hbm, idx_hbm, out_hbm, idx_v, rows_v, sem0, sem1, sem2):
        wid = lax.axis_index("s") * _NC + lax.axis_index("c")
        base = wid * b_per_w
        pltpu.sync_copy(idx_hbm.at[pl.ds(base, b_per_w)], idx_v)
        sems = (sem0, sem1, sem2)

        def start_gather(g):
            return pltpu.async_copy(
                table_hbm.at[idx_v.at[pl.ds(g * _CHUNK, _CHUNK)]],
                rows_v.at[g % _NBUF],
                sems[g % _NBUF],
            )

        copies = {g: start_gather(g) for g in range(min(_NBUF, nchunk))}
        for g in range(nchunk):
            copies[g].wait()
            pltpu.sync_copy(
                rows_v.at[g % _NBUF], out_hbm.at[pl.ds(base + g * _CHUNK, _CHUNK)]
            )
            if g + _NBUF < nchunk:
                copies[g + _NBUF] = start_gather(g + _NBUF)

    return _gather_rows


_SM_ROWS = 1024


def _unpack_body(t_ref, o_ref):
    word_t = jnp.transpose(lax.bitcast_convert_type(t_ref[...], jnp.uint32))
    o_ref[:HALF, :] = lax.bitcast_convert_type(word_t << 16, jnp.float32)
    o_ref[HALF:, :] = lax.bitcast_convert_type(word_t, jnp.float32)[: DIM - HALF, :]


def _unpack_body_acc(acc_ref, t_ref, o_ref):
    del acc_ref
    _unpack_body(t_ref, o_ref)


def _unpack_part(rows_pk, acc, row0):
    bp = rows_pk.shape[0]
    steps = bp // _SM_ROWS
    col0 = row0 // _SM_ROWS
    rows_spec = pl.BlockSpec((_SM_ROWS, DIM_PK), lambda i: (i, 0))
    out_spec = pl.BlockSpec((DIM, _SM_ROWS), lambda i: (0, col0 + i))
    out_shape = jax.ShapeDtypeStruct((DIM, BATCH), jnp.float32)
    if acc is None:
        return pl.pallas_call(
            _unpack_body,
            grid=(steps,),
            in_specs=[rows_spec],
            out_specs=out_spec,
            out_shape=out_shape,
        )(rows_pk)
    return pl.pallas_call(
        _unpack_body_acc,
        grid=(steps,),
        in_specs=[pl.BlockSpec(memory_space=pl.ANY), rows_spec],
        out_specs=out_spec,
        out_shape=out_shape,
        input_output_aliases={0: 0},
    )(acc, rows_pk)


def kernel(x, table):
    table_pk = _pack_table(table)
    xi = x.astype(jnp.int32)
    parts = []
    row0 = 0
    for bp in _PARTS:
        parts.append(
            (_make_gather(bp)(table_pk, lax.slice(xi, (row0,), (row0 + bp,))), row0)
        )
        row0 += bp
    acc = None
    for rows_pk, r0 in parts:
        acc = _unpack_part(rows_pk, acc, r0)
    return jnp.transpose(acc)


# speedup vs baseline: 1.0185x; 1.0185x over previous
import functools

import jax
import jax.numpy as jnp
from jax import lax
from jax.experimental import pallas as pl
from jax.experimental.pallas import tpu as pltpu
from jax.experimental.pallas import tpu_sc as plsc

VOCAB = 1000
DIM = 1000
HALF = 512
DIM_PK = 512
BATCH = 16384

_PARTS = (9216, 7168)

_TBL_ROWS = 200


def _pack_body(t_ref, o_ref):
    t = t_ref[...]
    m = jnp.max(t, axis=1, keepdims=True)
    e = jnp.exp(t - m)
    sm = e / jnp.sum(e, axis=1, keepdims=True)
    lo = sm[:, :HALF]
    hi = jnp.concatenate(
        [sm[:, HALF:], jnp.zeros((_TBL_ROWS, 2 * HALF - DIM), jnp.float32)], axis=1
    )
    lo_b = lax.bitcast_convert_type(lo, jnp.uint32)
    hi_b = lax.bitcast_convert_type(hi, jnp.uint32)
    half = jnp.uint32(0x8000)
    word = ((hi_b + half) & jnp.uint32(0xFFFF0000)) | ((lo_b + half) >> 16)
    o_ref[...] = lax.bitcast_convert_type(word, jnp.float32)


def _pack_table(table):
    return pl.pallas_call(
        _pack_body,
        grid=(VOCAB // _TBL_ROWS,),
        in_specs=[pl.BlockSpec((_TBL_ROWS, DIM), lambda i: (i, 0))],
        out_specs=pl.BlockSpec((_TBL_ROWS, DIM_PK), lambda i: (i, 0)),
        out_shape=jax.ShapeDtypeStruct((VOCAB, DIM_PK), jnp.float32),
    )(table)


_NC = 2
_NS = 16
_NW = _NC * _NS
_CHUNK = 32
_NBUF = 3

_sc_mesh = plsc.VectorSubcoreMesh(core_axis_name="c", subcore_axis_name="s")


@functools.cache
def _make_gather(bp, row0):
    b_per_w = bp // _NW
    nchunk = b_per_w // _CHUNK

    @functools.partial(
        pl.kernel,
        out_type=jax.ShapeDtypeStruct((bp, DIM_PK), jnp.float32),
        mesh=_sc_mesh,
        scratch_types=[
            pltpu.VMEM((b_per_w,), jnp.int32),
            pltpu.VMEM((_NBUF, _CHUNK, DIM_PK), jnp.float32),
            pltpu.SemaphoreType.DMA,
            pltpu.SemaphoreType.DMA,
            pltpu.SemaphoreType.DMA,
        ],
    )
    def _gather_rows(table_hbm, idx_hbm, out_hbm, idx_v, rows_v, sem0, sem1, sem2):
        wid = lax.axis_index("s") * _NC + lax.axis_index("c")
        base = wid * b_per_w
        pltpu.sync_copy(idx_hbm.at[pl.ds(row0 + base, b_per_w)], idx_v)
        sems = (sem0, sem1, sem2)

        def start_gather(g):
            return pltpu.async_copy(
                table_hbm.at[idx_v.at[pl.ds(g * _CHUNK, _CHUNK)]],
                rows_v.at[g % _NBUF],
                sems[g % _NBUF],
            )

        copies = {g: start_gather(g) for g in range(min(_NBUF, nchunk))}
        for g in range(nchunk):
            copies[g].wait()
            pltpu.sync_copy(
                rows_v.at[g % _NBUF], out_hbm.at[pl.ds(base + g * _CHUNK, _CHUNK)]
            )
            if g + _NBUF < nchunk:
                copies[g + _NBUF] = start_gather(g + _NBUF)

    return _gather_rows


_SM_ROWS = 1024


def _unpack_body(t_ref, o_ref):
    word_t = jnp.transpose(lax.bitcast_convert_type(t_ref[...], jnp.uint32))
    o_ref[:HALF, :] = lax.bitcast_convert_type(word_t << 16, jnp.float32)
    o_ref[HALF:, :] = lax.bitcast_convert_type(word_t, jnp.float32)[: DIM - HALF, :]


def _unpack_body_acc(acc_ref, t_ref, o_ref):
    del acc_ref
    _unpack_body(t_ref, o_ref)


def _unpack_part(rows_pk, acc, row0):
    bp = rows_pk.shape[0]
    steps = bp // _SM_ROWS
    col0 = row0 // _SM_ROWS
    rows_spec = pl.BlockSpec((_SM_ROWS, DIM_PK), lambda i: (i, 0))
    out_spec = pl.BlockSpec((DIM, _SM_ROWS), lambda i: (0, col0 + i))
    out_shape = jax.ShapeDtypeStruct((DIM, BATCH), jnp.float32)
    if acc is None:
        return pl.pallas_call(
            _unpack_body,
            grid=(steps,),
            in_specs=[rows_spec],
            out_specs=out_spec,
            out_shape=out_shape,
        )(rows_pk)
    return pl.pallas_call(
        _unpack_body_acc,
        grid=(steps,),
        in_specs=[pl.BlockSpec(memory_space=pl.ANY), rows_spec],
        out_specs=out_spec,
        out_shape=out_shape,
        input_output_aliases={0: 0},
    )(acc, rows_pk)


def kernel(x, table):
    table_pk = _pack_table(table)
    xi = x.astype(jnp.int32)
    parts = []
    row0 = 0
    for bp in _PARTS:
        parts.append((_make_gather(bp, row0)(table_pk, xi), row0))
        row0 += bp
    acc = None
    for rows_pk, r0 in parts:
        acc = _unpack_part(rows_pk, acc, r0)
    return jnp.transpose(acc)


# speedup vs baseline: 1.2878x; 1.2644x over previous
import functools

import jax
import jax.numpy as jnp
from jax import lax
from jax.experimental import pallas as pl
from jax.experimental.pallas import tpu as pltpu
from jax.experimental.pallas import tpu_sc as plsc

VOCAB = 1000
DIM = 1000
HALF = 512
DIM_PK = 512
DIM_PAD = 1024
BATCH = 16384

_B_TC = 7168
_B_SC = BATCH - _B_TC

_TBL_ROWS = 1000


def _pack_body(t_ref, o_pk_ref, o_tb_ref):
    t = t_ref[...]
    m = jnp.max(t, axis=1, keepdims=True)
    e = jnp.exp(t - m)
    sm = e / jnp.sum(e, axis=1, keepdims=True)
    lo = sm[:, :HALF]
    hi = jnp.concatenate(
        [sm[:, HALF:], jnp.zeros((_TBL_ROWS, 2 * HALF - DIM), jnp.float32)], axis=1
    )
    lo_r = (lax.bitcast_convert_type(lo, jnp.uint32) + jnp.uint32(0x8000)) >> 16
    hi_r = (lax.bitcast_convert_type(hi, jnp.uint32) + jnp.uint32(0x8000)) >> 16
    o_pk_ref[...] = lax.bitcast_convert_type((hi_r << 16) | lo_r, jnp.float32)
    sm_bf = jnp.concatenate(
        [
            lax.bitcast_convert_type(lo_r.astype(jnp.uint16), jnp.bfloat16),
            lax.bitcast_convert_type(hi_r.astype(jnp.uint16), jnp.bfloat16),
        ],
        axis=1,
    )
    o_tb_ref[...] = jnp.transpose(sm_bf)


def _pack_table(table):
    return pl.pallas_call(
        _pack_body,
        out_shape=[
            jax.ShapeDtypeStruct((VOCAB, DIM_PK), jnp.float32),
            jax.ShapeDtypeStruct((DIM_PAD, VOCAB), jnp.bfloat16),
        ],
    )(table)


_NC = 2
_NS = 16
_NW = _NC * _NS
_B_PER_W = _B_SC // _NW
_CHUNK = 32
_NCHUNK = _B_PER_W // _CHUNK
_NBUF = 3

_sc_mesh = plsc.VectorSubcoreMesh(core_axis_name="c", subcore_axis_name="s")


@functools.partial(
    pl.kernel,
    out_type=jax.ShapeDtypeStruct((_B_SC, DIM_PK), jnp.float32),
    mesh=_sc_mesh,
    scratch_types=[
        pltpu.VMEM((_B_PER_W,), jnp.int32),
        pltpu.VMEM((_NBUF, _CHUNK, DIM_PK), jnp.float32),
        pltpu.SemaphoreType.DMA,
        pltpu.SemaphoreType.DMA,
        pltpu.SemaphoreType.DMA,
    ],
)
def _gather_rows(table_hbm, idx_hbm, out_hbm, idx_v, rows_v, sem0, sem1, sem2):
    wid = lax.axis_index("s") * _NC + lax.axis_index("c")
    base = wid * _B_PER_W
    pltpu.sync_copy(idx_hbm.at[pl.ds(_B_TC + base, _B_PER_W)], idx_v)
    sems = (sem0, sem1, sem2)

    def start_gather(g):
        return pltpu.async_copy(
            table_hbm.at[idx_v.at[pl.ds(g * _CHUNK, _CHUNK)]],
            rows_v.at[g % _NBUF],
            sems[g % _NBUF],
        )

    copies = {g: start_gather(g) for g in range(min(_NBUF, _NCHUNK))}
    for g in range(_NCHUNK):
        copies[g].wait()
        pltpu.sync_copy(
            rows_v.at[g % _NBUF], out_hbm.at[pl.ds(base + g * _CHUNK, _CHUNK)]
        )
        if g + _NBUF < _NCHUNK:
            copies[g + _NBUF] = start_gather(g + _NBUF)


_MM_ROWS = 1024


def _mm_body(tb_ref, x_ref, o_ref):
    xb = x_ref[...]
    iota = lax.broadcasted_iota(jnp.int32, (VOCAB, _MM_ROWS), 0)
    onehot_t = (iota == xb[None, :]).astype(jnp.bfloat16)
    res = jnp.dot(tb_ref[...], onehot_t, preferred_element_type=jnp.float32)
    o_ref[...] = res[:DIM, :]


def _mm_part(table_bf, x):
    return pl.pallas_call(
        _mm_body,
        grid=(_B_TC // _MM_ROWS,),
        in_specs=[
            pl.BlockSpec((DIM_PAD, VOCAB), lambda i: (0, 0)),
            pl.BlockSpec((_MM_ROWS,), lambda i: (i,)),
        ],
        out_specs=pl.BlockSpec((DIM, _MM_ROWS), lambda i: (0, i)),
        out_shape=jax.ShapeDtypeStruct((DIM, BATCH), jnp.float32),
    )(table_bf, x)


_SM_ROWS = 1024


def _unpack_body(acc_ref, t_ref, o_ref):
    del acc_ref
    word_t = jnp.transpose(lax.bitcast_convert_type(t_ref[...], jnp.uint32))
    o_ref[:HALF, :] = lax.bitcast_convert_type(word_t << 16, jnp.float32)
    o_ref[HALF:, :] = lax.bitcast_convert_type(word_t, jnp.float32)[: DIM - HALF, :]


def _unpack_part(rows_pk, acc):
    col0 = _B_TC // _SM_ROWS
    return pl.pallas_call(
        _unpack_body,
        grid=(_B_SC // _SM_ROWS,),
        in_specs=[
            pl.BlockSpec(memory_space=pl.ANY),
            pl.BlockSpec((_SM_ROWS, DIM_PK), lambda i: (i, 0)),
        ],
        out_specs=pl.BlockSpec((DIM, _SM_ROWS), lambda i: (0, col0 + i)),
        out_shape=jax.ShapeDtypeStruct((DIM, BATCH), jnp.float32),
        input_output_aliases={0: 0},
    )(acc, rows_pk)


def kernel(x, table):
    table_pk, table_bf = _pack_table(table)
    xi = x.astype(jnp.int32)
    rows_pk = _gather_rows(table_pk, xi)
    acc = _mm_part(table_bf, xi)
    acc = _unpack_part(rows_pk, acc)
    return jnp.transpose(acc)


# speedup vs baseline: 1.2995x; 1.0091x over previous
import functools

import jax
import jax.numpy as jnp
from jax import lax
from jax.experimental import pallas as pl
from jax.experimental.pallas import tpu as pltpu
from jax.experimental.pallas import tpu_sc as plsc

VOCAB = 1000
DIM = 1000
HALF = 512
DIM_PK = 512
DIM_PAD = 1024
BATCH = 16384

_B_TC = 8192
_B_SC = BATCH - _B_TC

_TBL_ROWS = 1000


def _pack_body(t_ref, o_pk_ref, o_tb_ref):
    t = t_ref[...]
    m = jnp.max(t, axis=1, keepdims=True)
    e = jnp.exp(t - m)
    sm = e / jnp.sum(e, axis=1, keepdims=True)
    lo = sm[:, :HALF]
    hi = jnp.concatenate(
        [sm[:, HALF:], jnp.zeros((_TBL_ROWS, 2 * HALF - DIM), jnp.float32)], axis=1
    )
    lo_r = (lax.bitcast_convert_type(lo, jnp.uint32) + jnp.uint32(0x8000)) >> 16
    hi_r = (lax.bitcast_convert_type(hi, jnp.uint32) + jnp.uint32(0x8000)) >> 16
    o_pk_ref[...] = lax.bitcast_convert_type((hi_r << 16) | lo_r, jnp.float32)
    sm_bf = jnp.concatenate(
        [
            lax.bitcast_convert_type(lo_r.astype(jnp.uint16), jnp.bfloat16),
            lax.bitcast_convert_type(hi_r.astype(jnp.uint16), jnp.bfloat16),
        ],
        axis=1,
    )
    o_tb_ref[...] = jnp.transpose(sm_bf)


def _pack_table(table):
    return pl.pallas_call(
        _pack_body,
        out_shape=[
            jax.ShapeDtypeStruct((VOCAB, DIM_PK), jnp.float32),
            jax.ShapeDtypeStruct((DIM_PAD, VOCAB), jnp.bfloat16),
        ],
    )(table)


_NC = 2
_NS = 16
_NW = _NC * _NS
_B_PER_W = _B_SC // _NW
_CHUNK = 32
_NCHUNK = _B_PER_W // _CHUNK
_NBUF = 3

_sc_mesh = plsc.VectorSubcoreMesh(core_axis_name="c", subcore_axis_name="s")


@functools.partial(
    pl.kernel,
    out_type=jax.ShapeDtypeStruct((_B_SC, DIM_PK), jnp.float32),
    mesh=_sc_mesh,
    scratch_types=[
        pltpu.VMEM((_B_PER_W,), jnp.int32),
        pltpu.VMEM((_NBUF, _CHUNK, DIM_PK), jnp.float32),
        pltpu.SemaphoreType.DMA,
        pltpu.SemaphoreType.DMA,
        pltpu.SemaphoreType.DMA,
    ],
)
def _gather_rows(table_hbm, idx_hbm, out_hbm, idx_v, rows_v, sem0, sem1, sem2):
    wid = lax.axis_index("s") * _NC + lax.axis_index("c")
    base = wid * _B_PER_W
    pltpu.sync_copy(idx_hbm.at[pl.ds(_B_TC + base, _B_PER_W)], idx_v)
    sems = (sem0, sem1, sem2)

    def start_gather(g):
        return pltpu.async_copy(
            table_hbm.at[idx_v.at[pl.ds(g * _CHUNK, _CHUNK)]],
            rows_v.at[g % _NBUF],
            sems[g % _NBUF],
        )

    copies = {g: start_gather(g) for g in range(min(_NBUF, _NCHUNK))}
    for g in range(_NCHUNK):
        copies[g].wait()
        pltpu.sync_copy(
            rows_v.at[g % _NBUF], out_hbm.at[pl.ds(base + g * _CHUNK, _CHUNK)]
        )
        if g + _NBUF < _NCHUNK:
            copies[g + _NBUF] = start_gather(g + _NBUF)


_MM_ROWS = 1024


def _mm_body(tb_ref, x_ref, o_ref):
    xb = x_ref[...]
    iota = lax.broadcasted_iota(jnp.int32, (VOCAB, _MM_ROWS), 0)
    onehot_t = (iota == xb[None, :]).astype(jnp.bfloat16)
    res = jnp.dot(tb_ref[...], onehot_t, preferred_element_type=jnp.float32)
    o_ref[...] = res[:DIM, :]


def _mm_part(table_bf, x):
    return pl.pallas_call(
        _mm_body,
        grid=(_B_TC // _MM_ROWS,),
        in_specs=[
            pl.BlockSpec((DIM_PAD, VOCAB), lambda i: (0, 0)),
            pl.BlockSpec((_MM_ROWS,), lambda i: (i,)),
        ],
        out_specs=pl.BlockSpec((DIM, _MM_ROWS), lambda i: (0, i)),
        out_shape=jax.ShapeDtypeStruct((DIM, BATCH), jnp.float32),
    )(table_bf, x)


_SM_ROWS = 1024


def _unpack_body(acc_ref, t_ref, o_ref):
    del acc_ref
    word_t = jnp.transpose(lax.bitcast_convert_type(t_ref[...], jnp.uint32))
    o_ref[:HALF, :] = lax.bitcast_convert_type(word_t << 16, jnp.float32)
    o_ref[HALF:, :] = lax.bitcast_convert_type(word_t, jnp.float32)[: DIM - HALF, :]


def _unpack_part(rows_pk, acc):
    col0 = _B_TC // _SM_ROWS
    return pl.pallas_call(
        _unpack_body,
        grid=(_B_SC // _SM_ROWS,),
        in_specs=[
            pl.BlockSpec(memory_space=pl.ANY),
            pl.BlockSpec((_SM_ROWS, DIM_PK), lambda i: (i, 0)),
        ],
        out_specs=pl.BlockSpec((DIM, _SM_ROWS), lambda i: (0, col0 + i)),
        out_shape=jax.ShapeDtypeStruct((DIM, BATCH), jnp.float32),
        input_output_aliases={0: 0},
    )(acc, rows_pk)


def kernel(x, table):
    table_pk, table_bf = _pack_table(table)
    xi = x.astype(jnp.int32)
    rows_pk = _gather_rows(table_pk, xi)
    acc = _mm_part(table_bf, xi)
    acc = _unpack_part(rows_pk, acc)
    return jnp.transpose(acc)


# speedup vs baseline: 1.3013x; 1.0014x over previous
import functools

import jax
import jax.numpy as jnp
from jax import lax
from jax.experimental import pallas as pl
from jax.experimental.pallas import tpu as pltpu
from jax.experimental.pallas import tpu_sc as plsc

VOCAB = 1000
DIM = 1000
HALF = 512
DIM_PK = 512
DIM_PAD = 1024
BATCH = 16384

_B_TC = 9216
_B_SC = BATCH - _B_TC

_TBL_ROWS = 1000


def _pack_body(t_ref, o_pk_ref, o_tb_ref):
    t = t_ref[...]
    m = jnp.max(t, axis=1, keepdims=True)
    e = jnp.exp(t - m)
    sm = e / jnp.sum(e, axis=1, keepdims=True)
    lo = sm[:, :HALF]
    hi = jnp.concatenate(
        [sm[:, HALF:], jnp.zeros((_TBL_ROWS, 2 * HALF - DIM), jnp.float32)], axis=1
    )
    lo_r = (lax.bitcast_convert_type(lo, jnp.uint32) + jnp.uint32(0x8000)) >> 16
    hi_r = (lax.bitcast_convert_type(hi, jnp.uint32) + jnp.uint32(0x8000)) >> 16
    o_pk_ref[...] = lax.bitcast_convert_type((hi_r << 16) | lo_r, jnp.float32)
    sm_bf = jnp.concatenate(
        [
            lax.bitcast_convert_type(lo_r.astype(jnp.uint16), jnp.bfloat16),
            lax.bitcast_convert_type(hi_r.astype(jnp.uint16), jnp.bfloat16),
        ],
        axis=1,
    )
    o_tb_ref[...] = jnp.transpose(sm_bf)


def _pack_table(table):
    return pl.pallas_call(
        _pack_body,
        out_shape=[
            jax.ShapeDtypeStruct((VOCAB, DIM_PK), jnp.float32),
            jax.ShapeDtypeStruct((DIM_PAD, VOCAB), jnp.bfloat16),
        ],
    )(table)


_NC = 2
_NS = 16
_NW = _NC * _NS
_B_PER_W = _B_SC // _NW
_CHUNK = 32
_NCHUNK = _B_PER_W // _CHUNK
_NBUF = 3

_sc_mesh = plsc.VectorSubcoreMesh(core_axis_name="c", subcore_axis_name="s")


@functools.partial(
    pl.kernel,
    out_type=jax.ShapeDtypeStruct((_B_SC, DIM_PK), jnp.float32),
    mesh=_sc_mesh,
    scratch_types=[
        pltpu.VMEM((_B_PER_W,), jnp.int32),
        pltpu.VMEM((_NBUF, _CHUNK, DIM_PK), jnp.float32),
        pltpu.SemaphoreType.DMA,
        pltpu.SemaphoreType.DMA,
        pltpu.SemaphoreType.DMA,
    ],
)
def _gather_rows(table_hbm, idx_hbm, out_hbm, idx_v, rows_v, sem0, sem1, sem2):
    wid = lax.axis_index("s") * _NC + lax.axis_index("c")
    base = wid * _B_PER_W
    pltpu.sync_copy(idx_hbm.at[pl.ds(_B_TC + base, _B_PER_W)], idx_v)
    sems = (sem0, sem1, sem2)

    def start_gather(g):
        return pltpu.async_copy(
            table_hbm.at[idx_v.at[pl.ds(g * _CHUNK, _CHUNK)]],
            rows_v.at[g % _NBUF],
            sems[g % _NBUF],
        )

    copies = {g: start_gather(g) for g in range(min(_NBUF, _NCHUNK))}
    for g in range(_NCHUNK):
        copies[g].wait()
        pltpu.sync_copy(
            rows_v.at[g % _NBUF], out_hbm.at[pl.ds(base + g * _CHUNK, _CHUNK)]
        )
        if g + _NBUF < _NCHUNK:
            copies[g + _NBUF] = start_gather(g + _NBUF)


_MM_ROWS = 1024


def _mm_body(tb_ref, x_ref, o_ref):
    xb = x_ref[...]
    iota = lax.broadcasted_iota(jnp.int32, (VOCAB, _MM_ROWS), 0)
    onehot_t = (iota == xb[None, :]).astype(jnp.bfloat16)
    res = jnp.dot(tb_ref[...], onehot_t, preferred_element_type=jnp.float32)
    o_ref[...] = res[:DIM, :]


def _mm_part(table_bf, x):
    return pl.pallas_call(
        _mm_body,
        grid=(_B_TC // _MM_ROWS,),
        in_specs=[
            pl.BlockSpec((DIM_PAD, VOCAB), lambda i: (0, 0)),
            pl.BlockSpec((_MM_ROWS,), lambda i: (i,)),
        ],
        out_specs=pl.BlockSpec((DIM, _MM_ROWS), lambda i: (0, i)),
        out_shape=jax.ShapeDtypeStruct((DIM, BATCH), jnp.float32),
    )(table_bf, x)


_SM_ROWS = 1024


def _unpack_body(acc_ref, t_ref, o_ref):
    del acc_ref
    word_t = jnp.transpose(lax.bitcast_convert_type(t_ref[...], jnp.uint32))
    o_ref[:HALF, :] = lax.bitcast_convert_type(word_t << 16, jnp.float32)
    o_ref[HALF:, :] = lax.bitcast_convert_type(word_t, jnp.float32)[: DIM - HALF, :]


def _unpack_part(rows_pk, acc):
    col0 = _B_TC // _SM_ROWS
    return pl.pallas_call(
        _unpack_body,
        grid=(_B_SC // _SM_ROWS,),
        in_specs=[
            pl.BlockSpec(memory_space=pl.ANY),
            pl.BlockSpec((_SM_ROWS, DIM_PK), lambda i: (i, 0)),
        ],
        out_specs=pl.BlockSpec((DIM, _SM_ROWS), lambda i: (0, col0 + i)),
        out_shape=jax.ShapeDtypeStruct((DIM, BATCH), jnp.float32),
        input_output_aliases={0: 0},
    )(acc, rows_pk)


def kernel(x, table):
    table_pk, table_bf = _pack_table(table)
    xi = x.astype(jnp.int32)
    rows_pk = _gather_rows(table_pk, xi)
    acc = _mm_part(table_bf, xi)
    acc = _unpack_part(rows_pk, acc)
    return jnp.transpose(acc)


# speedup vs baseline: 1.3265x; 1.0194x over previous
import functools

import jax
import jax.numpy as jnp
from jax import lax
from jax.experimental import pallas as pl
from jax.experimental.pallas import tpu as pltpu
from jax.experimental.pallas import tpu_sc as plsc

VOCAB = 1000
DIM = 1000
HALF = 512
DIM_PK = 512
DIM_PAD = 1024
BATCH = 16384

_B_TC = 10240
_B_SC = BATCH - _B_TC

_TBL_ROWS = 1000


def _pack_body(t_ref, o_pk_ref, o_tb_ref):
    t = t_ref[...]
    m = jnp.max(t, axis=1, keepdims=True)
    e = jnp.exp(t - m)
    sm = e / jnp.sum(e, axis=1, keepdims=True)
    lo = sm[:, :HALF]
    hi = jnp.concatenate(
        [sm[:, HALF:], jnp.zeros((_TBL_ROWS, 2 * HALF - DIM), jnp.float32)], axis=1
    )
    lo_r = (lax.bitcast_convert_type(lo, jnp.uint32) + jnp.uint32(0x8000)) >> 16
    hi_r = (lax.bitcast_convert_type(hi, jnp.uint32) + jnp.uint32(0x8000)) >> 16
    o_pk_ref[...] = lax.bitcast_convert_type((hi_r << 16) | lo_r, jnp.float32)
    sm_bf = jnp.concatenate(
        [
            lax.bitcast_convert_type(lo_r.astype(jnp.uint16), jnp.bfloat16),
            lax.bitcast_convert_type(hi_r.astype(jnp.uint16), jnp.bfloat16),
        ],
        axis=1,
    )
    o_tb_ref[...] = jnp.transpose(sm_bf)


def _pack_table(table):
    return pl.pallas_call(
        _pack_body,
        out_shape=[
            jax.ShapeDtypeStruct((VOCAB, DIM_PK), jnp.float32),
            jax.ShapeDtypeStruct((DIM_PAD, VOCAB), jnp.bfloat16),
        ],
    )(table)


_NC = 2
_NS = 16
_NW = _NC * _NS
_B_PER_W = _B_SC // _NW
_CHUNK = 32
_NCHUNK = _B_PER_W // _CHUNK
_NBUF = 3

_sc_mesh = plsc.VectorSubcoreMesh(core_axis_name="c", subcore_axis_name="s")


@functools.partial(
    pl.kernel,
    out_type=jax.ShapeDtypeStruct((_B_SC, DIM_PK), jnp.float32),
    mesh=_sc_mesh,
    scratch_types=[
        pltpu.VMEM((_B_PER_W,), jnp.int32),
        pltpu.VMEM((_NBUF, _CHUNK, DIM_PK), jnp.float32),
        pltpu.SemaphoreType.DMA,
        pltpu.SemaphoreType.DMA,
        pltpu.SemaphoreType.DMA,
    ],
)
def _gather_rows(table_hbm, idx_hbm, out_hbm, idx_v, rows_v, sem0, sem1, sem2):
    wid = lax.axis_index("s") * _NC + lax.axis_index("c")
    base = wid * _B_PER_W
    pltpu.sync_copy(idx_hbm.at[pl.ds(_B_TC + base, _B_PER_W)], idx_v)
    sems = (sem0, sem1, sem2)

    def start_gather(g):
        return pltpu.async_copy(
            table_hbm.at[idx_v.at[pl.ds(g * _CHUNK, _CHUNK)]],
            rows_v.at[g % _NBUF],
            sems[g % _NBUF],
        )

    copies = {g: start_gather(g) for g in range(min(_NBUF, _NCHUNK))}
    for g in range(_NCHUNK):
        copies[g].wait()
        pltpu.sync_copy(
            rows_v.at[g % _NBUF], out_hbm.at[pl.ds(base + g * _CHUNK, _CHUNK)]
        )
        if g + _NBUF < _NCHUNK:
            copies[g + _NBUF] = start_gather(g + _NBUF)


_MM_ROWS = 1024


def _mm_body(tb_ref, x_ref, o_ref):
    xb = x_ref[...]
    iota = lax.broadcasted_iota(jnp.int32, (VOCAB, _MM_ROWS), 0)
    onehot_t = (iota == xb[None, :]).astype(jnp.bfloat16)
    res = jnp.dot(tb_ref[...], onehot_t, preferred_element_type=jnp.float32)
    o_ref[...] = res[:DIM, :]


def _mm_part(table_bf, x):
    return pl.pallas_call(
        _mm_body,
        grid=(_B_TC // _MM_ROWS,),
        in_specs=[
            pl.BlockSpec((DIM_PAD, VOCAB), lambda i: (0, 0)),
            pl.BlockSpec((_MM_ROWS,), lambda i: (i,)),
        ],
        out_specs=pl.BlockSpec((DIM, _MM_ROWS), lambda i: (0, i)),
        out_shape=jax.ShapeDtypeStruct((DIM, BATCH), jnp.float32),
    )(table_bf, x)


_SM_ROWS = 1024


def _unpack_body(acc_ref, t_ref, o_ref):
    del acc_ref
    word_t = jnp.transpose(lax.bitcast_convert_type(t_ref[...], jnp.uint32))
    o_ref[:HALF, :] = lax.bitcast_convert_type(word_t << 16, jnp.float32)
    o_ref[HALF:, :] = lax.bitcast_convert_type(word_t, jnp.float32)[: DIM - HALF, :]


def _unpack_part(rows_pk, acc):
    col0 = _B_TC // _SM_ROWS
    return pl.pallas_call(
        _unpack_body,
        grid=(_B_SC // _SM_ROWS,),
        in_specs=[
            pl.BlockSpec(memory_space=pl.ANY),
            pl.BlockSpec((_SM_ROWS, DIM_PK), lambda i: (i, 0)),
        ],
        out_specs=pl.BlockSpec((DIM, _SM_ROWS), lambda i: (0, col0 + i)),
        out_shape=jax.ShapeDtypeStruct((DIM, BATCH), jnp.float32),
        input_output_aliases={0: 0},
    )(acc, rows_pk)


def kernel(x, table):
    table_pk, table_bf = _pack_table(table)
    xi = x.astype(jnp.int32)
    rows_pk = _gather_rows(table_pk, xi)
    acc = _mm_part(table_bf, xi)
    acc = _unpack_part(rows_pk, acc)
    return jnp.transpose(acc)


# speedup vs baseline: 1.3289x; 1.0018x over previous
import functools

import jax
import jax.numpy as jnp
from jax import lax
from jax.experimental import pallas as pl
from jax.experimental.pallas import tpu as pltpu
from jax.experimental.pallas import tpu_sc as plsc

VOCAB = 1000
DIM = 1000
HALF = 512
DIM_PK = 512
DIM_PAD = 1024
BATCH = 16384

_B_TC = 11264
_B_SC = BATCH - _B_TC

_TBL_ROWS = 1000


def _pack_body(t_ref, o_pk_ref, o_tb_ref):
    t = t_ref[...]
    m = jnp.max(t, axis=1, keepdims=True)
    e = jnp.exp(t - m)
    sm = e / jnp.sum(e, axis=1, keepdims=True)
    lo = sm[:, :HALF]
    hi = jnp.concatenate(
        [sm[:, HALF:], jnp.zeros((_TBL_ROWS, 2 * HALF - DIM), jnp.float32)], axis=1
    )
    lo_r = (lax.bitcast_convert_type(lo, jnp.uint32) + jnp.uint32(0x8000)) >> 16
    hi_r = (lax.bitcast_convert_type(hi, jnp.uint32) + jnp.uint32(0x8000)) >> 16
    o_pk_ref[...] = lax.bitcast_convert_type((hi_r << 16) | lo_r, jnp.float32)
    sm_bf = jnp.concatenate(
        [
            lax.bitcast_convert_type(lo_r.astype(jnp.uint16), jnp.bfloat16),
            lax.bitcast_convert_type(hi_r.astype(jnp.uint16), jnp.bfloat16),
        ],
        axis=1,
    )
    o_tb_ref[...] = jnp.transpose(sm_bf)


def _pack_table(table):
    return pl.pallas_call(
        _pack_body,
        out_shape=[
            jax.ShapeDtypeStruct((VOCAB, DIM_PK), jnp.float32),
            jax.ShapeDtypeStruct((DIM_PAD, VOCAB), jnp.bfloat16),
        ],
    )(table)


_NC = 2
_NS = 16
_NW = _NC * _NS
_B_PER_W = _B_SC // _NW
_CHUNK = 32
_NCHUNK = _B_PER_W // _CHUNK
_NBUF = 3

_sc_mesh = plsc.VectorSubcoreMesh(core_axis_name="c", subcore_axis_name="s")


@functools.partial(
    pl.kernel,
    out_type=jax.ShapeDtypeStruct((_B_SC, DIM_PK), jnp.float32),
    mesh=_sc_mesh,
    scratch_types=[
        pltpu.VMEM((_B_PER_W,), jnp.int32),
        pltpu.VMEM((_NBUF, _CHUNK, DIM_PK), jnp.float32),
        pltpu.SemaphoreType.DMA,
        pltpu.SemaphoreType.DMA,
        pltpu.SemaphoreType.DMA,
    ],
)
def _gather_rows(table_hbm, idx_hbm, out_hbm, idx_v, rows_v, sem0, sem1, sem2):
    wid = lax.axis_index("s") * _NC + lax.axis_index("c")
    base = wid * _B_PER_W
    pltpu.sync_copy(idx_hbm.at[pl.ds(_B_TC + base, _B_PER_W)], idx_v)
    sems = (sem0, sem1, sem2)

    def start_gather(g):
        return pltpu.async_copy(
            table_hbm.at[idx_v.at[pl.ds(g * _CHUNK, _CHUNK)]],
            rows_v.at[g % _NBUF],
            sems[g % _NBUF],
        )

    copies = {g: start_gather(g) for g in range(min(_NBUF, _NCHUNK))}
    for g in range(_NCHUNK):
        copies[g].wait()
        pltpu.sync_copy(
            rows_v.at[g % _NBUF], out_hbm.at[pl.ds(base + g * _CHUNK, _CHUNK)]
        )
        if g + _NBUF < _NCHUNK:
            copies[g + _NBUF] = start_gather(g + _NBUF)


_MM_ROWS = 1024


def _mm_body(tb_ref, x_ref, o_ref):
    xb = x_ref[...]
    iota = lax.broadcasted_iota(jnp.int32, (VOCAB, _MM_ROWS), 0)
    onehot_t = (iota == xb[None, :]).astype(jnp.bfloat16)
    res = jnp.dot(tb_ref[...], onehot_t, preferred_element_type=jnp.float32)
    o_ref[...] = res[:DIM, :]


def _mm_part(table_bf, x):
    return pl.pallas_call(
        _mm_body,
        grid=(_B_TC // _MM_ROWS,),
        in_specs=[
            pl.BlockSpec((DIM_PAD, VOCAB), lambda i: (0, 0)),
            pl.BlockSpec((_MM_ROWS,), lambda i: (i,)),
        ],
        out_specs=pl.BlockSpec((DIM, _MM_ROWS), lambda i: (0, i)),
        out_shape=jax.ShapeDtypeStruct((DIM, BATCH), jnp.float32),
    )(table_bf, x)


_SM_ROWS = 1024


def _unpack_body(acc_ref, t_ref, o_ref):
    del acc_ref
    word_t = jnp.transpose(lax.bitcast_convert_type(t_ref[...], jnp.uint32))
    o_ref[:HALF, :] = lax.bitcast_convert_type(word_t << 16, jnp.float32)
    o_ref[HALF:, :] = lax.bitcast_convert_type(word_t, jnp.float32)[: DIM - HALF, :]


def _unpack_part(rows_pk, acc):
    col0 = _B_TC // _SM_ROWS
    return pl.pallas_call(
        _unpack_body,
        grid=(_B_SC // _SM_ROWS,),
        in_specs=[
            pl.BlockSpec(memory_space=pl.ANY),
            pl.BlockSpec((_SM_ROWS, DIM_PK), lambda i: (i, 0)),
        ],
        out_specs=pl.BlockSpec((DIM, _SM_ROWS), lambda i: (0, col0 + i)),
        out_shape=jax.ShapeDtypeStruct((DIM, BATCH), jnp.float32),
        input_output_aliases={0: 0},
    )(acc, rows_pk)


def kernel(x, table):
    table_pk, table_bf = _pack_table(table)
    xi = x.astype(jnp.int32)
    rows_pk = _gather_rows(table_pk, xi)
    acc = _mm_part(table_bf, xi)
    acc = _unpack_part(rows_pk, acc)
    return jnp.transpose(acc)


# speedup vs baseline: 1.3366x; 1.0058x over previous
import functools

import jax
import jax.numpy as jnp
from jax import lax
from jax.experimental import pallas as pl
from jax.experimental.pallas import tpu as pltpu
from jax.experimental.pallas import tpu_sc as plsc

VOCAB = 1000
DIM = 1000
HALF = 512
DIM_PK = 512
DIM_PAD = 1024
BATCH = 16384

_B_TC = 12288
_B_SC = BATCH - _B_TC

_TBL_ROWS = 1000


def _pack_body(t_ref, o_pk_ref, o_tb_ref):
    t = t_ref[...]
    m = jnp.max(t, axis=1, keepdims=True)
    e = jnp.exp(t - m)
    sm = e / jnp.sum(e, axis=1, keepdims=True)
    lo = sm[:, :HALF]
    hi = jnp.concatenate(
        [sm[:, HALF:], jnp.zeros((_TBL_ROWS, 2 * HALF - DIM), jnp.float32)], axis=1
    )
    lo_r = (lax.bitcast_convert_type(lo, jnp.uint32) + jnp.uint32(0x8000)) >> 16
    hi_r = (lax.bitcast_convert_type(hi, jnp.uint32) + jnp.uint32(0x8000)) >> 16
    o_pk_ref[...] = lax.bitcast_convert_type((hi_r << 16) | lo_r, jnp.float32)
    sm_bf = jnp.concatenate(
        [
            lax.bitcast_convert_type(lo_r.astype(jnp.uint16), jnp.bfloat16),
            lax.bitcast_convert_type(hi_r.astype(jnp.uint16), jnp.bfloat16),
        ],
        axis=1,
    )
    o_tb_ref[...] = jnp.transpose(sm_bf)


def _pack_table(table):
    return pl.pallas_call(
        _pack_body,
        out_shape=[
            jax.ShapeDtypeStruct((VOCAB, DIM_PK), jnp.float32),
            jax.ShapeDtypeStruct((DIM_PAD, VOCAB), jnp.bfloat16),
        ],
    )(table)


_NC = 2
_NS = 16
_NW = _NC * _NS
_B_PER_W = _B_SC // _NW
_CHUNK = 32
_NCHUNK = _B_PER_W // _CHUNK
_NBUF = 3

_sc_mesh = plsc.VectorSubcoreMesh(core_axis_name="c", subcore_axis_name="s")


@functools.partial(
    pl.kernel,
    out_type=jax.ShapeDtypeStruct((_B_SC, DIM_PK), jnp.float32),
    mesh=_sc_mesh,
    scratch_types=[
        pltpu.VMEM((_B_PER_W,), jnp.int32),
        pltpu.VMEM((_NBUF, _CHUNK, DIM_PK), jnp.float32),
        pltpu.SemaphoreType.DMA,
        pltpu.SemaphoreType.DMA,
        pltpu.SemaphoreType.DMA,
    ],
)
def _gather_rows(table_hbm, idx_hbm, out_hbm, idx_v, rows_v, sem0, sem1, sem2):
    wid = lax.axis_index("s") * _NC + lax.axis_index("c")
    base = wid * _B_PER_W
    pltpu.sync_copy(idx_hbm.at[pl.ds(_B_TC + base, _B_PER_W)], idx_v)
    sems = (sem0, sem1, sem2)

    def start_gather(g):
        return pltpu.async_copy(
            table_hbm.at[idx_v.at[pl.ds(g * _CHUNK, _CHUNK)]],
            rows_v.at[g % _NBUF],
            sems[g % _NBUF],
        )

    copies = {g: start_gather(g) for g in range(min(_NBUF, _NCHUNK))}
    for g in range(_NCHUNK):
        copies[g].wait()
        pltpu.sync_copy(
            rows_v.at[g % _NBUF], out_hbm.at[pl.ds(base + g * _CHUNK, _CHUNK)]
        )
        if g + _NBUF < _NCHUNK:
            copies[g + _NBUF] = start_gather(g + _NBUF)


_MM_ROWS = 1024


def _mm_body(tb_ref, x_ref, o_ref):
    xb = x_ref[...]
    iota = lax.broadcasted_iota(jnp.int32, (VOCAB, _MM_ROWS), 0)
    onehot_t = (iota == xb[None, :]).astype(jnp.bfloat16)
    res = jnp.dot(tb_ref[...], onehot_t, preferred_element_type=jnp.float32)
    o_ref[...] = res[:DIM, :]


def _mm_part(table_bf, x):
    return pl.pallas_call(
        _mm_body,
        grid=(_B_TC // _MM_ROWS,),
        in_specs=[
            pl.BlockSpec((DIM_PAD, VOCAB), lambda i: (0, 0)),
            pl.BlockSpec((_MM_ROWS,), lambda i: (i,)),
        ],
        out_specs=pl.BlockSpec((DIM, _MM_ROWS), lambda i: (0, i)),
        out_shape=jax.ShapeDtypeStruct((DIM, BATCH), jnp.float32),
    )(table_bf, x)


_SM_ROWS = 1024


def _unpack_body(acc_ref, t_ref, o_ref):
    del acc_ref
    word_t = jnp.transpose(lax.bitcast_convert_type(t_ref[...], jnp.uint32))
    o_ref[:HALF, :] = lax.bitcast_convert_type(word_t << 16, jnp.float32)
    o_ref[HALF:, :] = lax.bitcast_convert_type(word_t, jnp.float32)[: DIM - HALF, :]


def _unpack_part(rows_pk, acc):
    col0 = _B_TC // _SM_ROWS
    return pl.pallas_call(
        _unpack_body,
        grid=(_B_SC // _SM_ROWS,),
        in_specs=[
            pl.BlockSpec(memory_space=pl.ANY),
            pl.BlockSpec((_SM_ROWS, DIM_PK), lambda i: (i, 0)),
        ],
        out_specs=pl.BlockSpec((DIM, _SM_ROWS), lambda i: (0, col0 + i)),
        out_shape=jax.ShapeDtypeStruct((DIM, BATCH), jnp.float32),
        input_output_aliases={0: 0},
    )(acc, rows_pk)


def kernel(x, table):
    table_pk, table_bf = _pack_table(table)
    xi = x.astype(jnp.int32)
    rows_pk = _gather_rows(table_pk, xi)
    acc = _mm_part(table_bf, xi)
    acc = _unpack_part(rows_pk, acc)
    return jnp.transpose(acc)


# speedup vs baseline: 1.3943x; 1.0432x over previous
import functools

import jax
import jax.numpy as jnp
from jax import lax
from jax.experimental import pallas as pl
from jax.experimental.pallas import tpu as pltpu
from jax.experimental.pallas import tpu_sc as plsc

VOCAB = 1000
DIM = 1000
HALF = 512
DIM_PK = 512
DIM_PAD = 1024
BATCH = 16384

_B_TC = 12288
_B_SC = BATCH - _B_TC

_TBL_ROWS = 1000


def _pack_body(t_ref, o_pk_ref, o_tb_ref):
    t = t_ref[...]
    m = jnp.max(t, axis=1, keepdims=True)
    e = jnp.exp(t - m)
    sm = e / jnp.sum(e, axis=1, keepdims=True)
    lo = sm[:, :HALF]
    hi = jnp.concatenate(
        [sm[:, HALF:], jnp.zeros((_TBL_ROWS, 2 * HALF - DIM), jnp.float32)], axis=1
    )
    lo_r = (lax.bitcast_convert_type(lo, jnp.uint32) + jnp.uint32(0x8000)) >> 16
    hi_r = (lax.bitcast_convert_type(hi, jnp.uint32) + jnp.uint32(0x8000)) >> 16
    o_pk_ref[...] = lax.bitcast_convert_type((hi_r << 16) | lo_r, jnp.float32)
    sm_bf = jnp.concatenate(
        [
            lax.bitcast_convert_type(lo_r.astype(jnp.uint16), jnp.bfloat16),
            lax.bitcast_convert_type(hi_r.astype(jnp.uint16), jnp.bfloat16),
        ],
        axis=1,
    )
    o_tb_ref[...] = jnp.transpose(sm_bf)


def _pack_table(table):
    return pl.pallas_call(
        _pack_body,
        out_shape=[
            jax.ShapeDtypeStruct((VOCAB, DIM_PK), jnp.float32),
            jax.ShapeDtypeStruct((DIM_PAD, VOCAB), jnp.bfloat16),
        ],
    )(table)


_NC = 2
_NS = 16
_NW = _NC * _NS
_B_PER_W = _B_SC // _NW
_CHUNK = 32
_NCHUNK = _B_PER_W // _CHUNK
_NBUF = 3

_sc_mesh = plsc.VectorSubcoreMesh(core_axis_name="c", subcore_axis_name="s")


@functools.partial(
    pl.kernel,
    out_type=jax.ShapeDtypeStruct((_B_SC, DIM_PK), jnp.float32),
    mesh=_sc_mesh,
    scratch_types=[
        pltpu.VMEM((_B_PER_W,), jnp.int32),
        pltpu.VMEM((_NBUF, _CHUNK, DIM_PK), jnp.float32),
        pltpu.SemaphoreType.DMA,
        pltpu.SemaphoreType.DMA,
        pltpu.SemaphoreType.DMA,
    ],
)
def _gather_rows(table_hbm, idx_hbm, out_hbm, idx_v, rows_v, sem0, sem1, sem2):
    wid = lax.axis_index("s") * _NC + lax.axis_index("c")
    base = wid * _B_PER_W
    pltpu.sync_copy(idx_hbm.at[pl.ds(_B_TC + base, _B_PER_W)], idx_v)
    sems = (sem0, sem1, sem2)

    def start_gather(g):
        return pltpu.async_copy(
            table_hbm.at[idx_v.at[pl.ds(g * _CHUNK, _CHUNK)]],
            rows_v.at[g % _NBUF],
            sems[g % _NBUF],
        )

    copies = {g: start_gather(g) for g in range(min(_NBUF, _NCHUNK))}
    for g in range(_NCHUNK):
        copies[g].wait()
        pltpu.sync_copy(
            rows_v.at[g % _NBUF], out_hbm.at[pl.ds(base + g * _CHUNK, _CHUNK)]
        )
        if g + _NBUF < _NCHUNK:
            copies[g + _NBUF] = start_gather(g + _NBUF)


_MM_ROWS = 2048


def _mm_body(tb_ref, x_ref, o_ref):
    xb = x_ref[...]
    iota = lax.broadcasted_iota(jnp.int32, (VOCAB, _MM_ROWS), 0)
    onehot_t = (iota == xb[None, :]).astype(jnp.bfloat16)
    res = jnp.dot(tb_ref[...], onehot_t, preferred_element_type=jnp.float32)
    o_ref[...] = res[:DIM, :]


def _mm_part(table_bf, x):
    return pl.pallas_call(
        _mm_body,
        grid=(_B_TC // _MM_ROWS,),
        in_specs=[
            pl.BlockSpec((DIM_PAD, VOCAB), lambda i: (0, 0)),
            pl.BlockSpec((_MM_ROWS,), lambda i: (i,)),
        ],
        out_specs=pl.BlockSpec((DIM, _MM_ROWS), lambda i: (0, i)),
        out_shape=jax.ShapeDtypeStruct((DIM, BATCH), jnp.float32),
    )(table_bf, x)


_SM_ROWS = 2048


def _unpack_body(acc_ref, t_ref, o_ref):
    del acc_ref
    word_t = jnp.transpose(lax.bitcast_convert_type(t_ref[...], jnp.uint32))
    o_ref[:HALF, :] = lax.bitcast_convert_type(word_t << 16, jnp.float32)
    o_ref[HALF:, :] = lax.bitcast_convert_type(word_t, jnp.float32)[: DIM - HALF, :]


def _unpack_part(rows_pk, acc):
    col0 = _B_TC // _SM_ROWS
    return pl.pallas_call(
        _unpack_body,
        grid=(_B_SC // _SM_ROWS,),
        in_specs=[
            pl.BlockSpec(memory_space=pl.ANY),
            pl.BlockSpec((_SM_ROWS, DIM_PK), lambda i: (i, 0)),
        ],
        out_specs=pl.BlockSpec((DIM, _SM_ROWS), lambda i: (0, col0 + i)),
        out_shape=jax.ShapeDtypeStruct((DIM, BATCH), jnp.float32),
        input_output_aliases={0: 0},
    )(acc, rows_pk)


def kernel(x, table):
    table_pk, table_bf = _pack_table(table)
    xi = x.astype(jnp.int32)
    rows_pk = _gather_rows(table_pk, xi)
    acc = _mm_part(table_bf, xi)
    acc = _unpack_part(rows_pk, acc)
    return jnp.transpose(acc)
